# baseline (speedup 1.0000x reference)
import functools

import jax
import jax.numpy as jnp
from jax import lax
from jax.experimental import pallas as pl
from jax.experimental.pallas import tpu as pltpu
from jax.experimental.pallas import tpu_sc as plsc

_NC, _NS, _LANES = 2, 16, 16
_NW = _NC * _NS

_M, _N = 2048, 2048
_TOTAL = _M * _N
_ROWS_PW = _M // _NW
_CH = 8
_CHUNK = _CH * _N
_NCHUNK = _ROWS_PW // _CH
_GPR = _N // _LANES


_MAGIC = 12582912.0


def _rne(v):
    return (v + _MAGIC) - _MAGIC


def _fp4_group(a):
    lo = jnp.minimum(_rne(a + a), 4.0) * 0.5
    hi = jnp.where(a < 5.0, _rne(jnp.minimum(a, 4.0)), 6.0)
    return jnp.where(a < 2.5, lo, hi)


def _sc_body(x_hbm, s_hbm, o_hbm, s_v, inv_v, x_v, o_v):
    wid = lax.axis_index("s") * _NC + lax.axis_index("c")
    base = wid * (_ROWS_PW * _N)

    pltpu.sync_copy(s_hbm, s_v)
    for j in range(_GPR):
        inv_v[pl.ds(j * _LANES, _LANES)] = 1.0 / s_v[pl.ds(j * _LANES, _LANES)]

    def chunk_body(c, carry):
        cbase = base + c * _CHUNK
        pltpu.sync_copy(x_hbm.at[pl.ds(cbase, _CHUNK)], x_v)

        def row_body(r, carry2):
            rb = r * _N
            for j in range(_GPR):
                off = j * _LANES
                xv = x_v[pl.ds(rb + off, _LANES)]
                inv = inv_v[pl.ds(off, _LANES)]
                sv = s_v[pl.ds(off, _LANES)]
                a = jnp.abs(xv) * inv
                mag = _fp4_group(a) * sv
                o_v[pl.ds(rb + off, _LANES)] = jnp.where(xv < 0.0, -mag, mag)
            return carry2

        lax.fori_loop(0, _CH, row_body, 0)
        pltpu.sync_copy(o_v, o_hbm.at[pl.ds(cbase, _CHUNK)])
        return carry

    lax.fori_loop(0, _NCHUNK, chunk_body, 0)


def _sc_quantize(x2, s1):
    mesh = plsc.VectorSubcoreMesh(core_axis_name="c", subcore_axis_name="s")
    k = functools.partial(
        pl.kernel,
        out_type=jax.ShapeDtypeStruct((_TOTAL,), jnp.float32),
        mesh=mesh,
        scratch_types=[
            pltpu.VMEM((_N,), jnp.float32),
            pltpu.VMEM((_N,), jnp.float32),
            pltpu.VMEM((_CHUNK,), jnp.float32),
            pltpu.VMEM((_CHUNK,), jnp.float32),
        ],
    )(_sc_body)
    return k(x2.reshape(_TOTAL), s1.reshape(_N))


def kernel(x, scale, code):
    del code
    B, M, N = x.shape
    out = _sc_quantize(x.reshape(B * M, N), scale.reshape(N))
    return out.reshape(B, M, N)


# speedup vs baseline: 2.9481x; 2.9481x over previous
import functools

import jax
import jax.numpy as jnp
from jax import lax
from jax.experimental import pallas as pl
from jax.experimental.pallas import tpu as pltpu
from jax.experimental.pallas import tpu_sc as plsc

_NC, _NS, _LANES = 2, 16, 16
_NW = _NC * _NS

_M, _N = 2048, 2048
_TOTAL = _M * _N
_ROWS_PW = _M // _NW
_CH = 8
_CHUNK = _CH * _N
_NCHUNK = _ROWS_PW // _CH
_GPR = _N // _LANES


_MAGIC = 12582912.0


def _rne(v):
    return (v + _MAGIC) - _MAGIC


def _fp4_group(a):
    lo = jnp.minimum(_rne(a + a), 4.0) * 0.5
    hi = jnp.where(a < 5.0, _rne(jnp.minimum(a, 4.0)), 6.0)
    return jnp.where(a < 2.5, lo, hi)


def _sc_body(x_hbm, s_hbm, o_hbm, s_v, inv_v, x_v, o_v):
    wid = lax.axis_index("s") * _NC + lax.axis_index("c")
    base = wid * (_ROWS_PW * _N)

    pltpu.sync_copy(s_hbm, s_v)
    for j in range(_GPR):
        inv_v[pl.ds(j * _LANES, _LANES)] = 1.0 / s_v[pl.ds(j * _LANES, _LANES)]

    def chunk_body(c, carry):
        cbase = base + c * _CHUNK
        pltpu.sync_copy(x_hbm.at[pl.ds(cbase, _CHUNK)], x_v)

        def group_body(j, carry2):
            off = j * _LANES
            inv = inv_v[pl.ds(off, _LANES)]
            sv = s_v[pl.ds(off, _LANES)]
            for r in range(_CH):
                idx = r * _N + off
                xv = x_v[pl.ds(idx, _LANES)]
                a = jnp.abs(xv) * inv
                mag = _fp4_group(a) * sv
                o_v[pl.ds(idx, _LANES)] = jnp.where(xv < 0.0, -mag, mag)
            return carry2

        lax.fori_loop(0, _GPR, group_body, 0)
        pltpu.sync_copy(o_v, o_hbm.at[pl.ds(cbase, _CHUNK)])
        return carry

    lax.fori_loop(0, _NCHUNK, chunk_body, 0)


def _sc_quantize(x2, s1):
    mesh = plsc.VectorSubcoreMesh(core_axis_name="c", subcore_axis_name="s")
    k = functools.partial(
        pl.kernel,
        out_type=jax.ShapeDtypeStruct((_TOTAL,), jnp.float32),
        mesh=mesh,
        scratch_types=[
            pltpu.VMEM((_N,), jnp.float32),
            pltpu.VMEM((_N,), jnp.float32),
            pltpu.VMEM((_CHUNK,), jnp.float32),
            pltpu.VMEM((_CHUNK,), jnp.float32),
        ],
    )(_sc_body)
    return k(x2.reshape(_TOTAL), s1.reshape(_N))


def kernel(x, scale, code):
    del code
    B, M, N = x.shape
    out = _sc_quantize(x.reshape(B * M, N), scale.reshape(N))
    return out.reshape(B, M, N)
